# Initial kernel scaffold; baseline (speedup 1.0000x reference)
import jax
import jax.numpy as jnp
from jax.experimental import pallas as pl


def kernel(x, edge_index, W1, b1, W2, b2, W3, b3, Wc1, bc1, Wc2, bc2, Wf1, bf1, Wf2, bf2):
    raise NotImplementedError("write your pallas kernel here")


# speedup vs baseline: 13.3824x; 13.3824x over previous
import functools

import jax
import jax.numpy as jnp
from jax import lax
from jax.experimental import pallas as pl
from jax.experimental.pallas import tpu as pltpu
from jax.experimental.pallas import tpu_sc as plsc

_K = 128
_NW = 32
_R = 1024
_DW = 16

_HIGH = jax.lax.Precision.HIGHEST


@functools.lru_cache(maxsize=None)
def _sc_agg(n_pad: int, e_pad: int, h: int, gather: bool):
    ept = e_pad // _NW
    nchunks = ept // _K
    rps = n_pad // 16
    mesh = plsc.VectorSubcoreMesh(core_axis_name="c", subcore_axis_name="s")

    @functools.partial(
        pl.kernel,
        out_type=jax.ShapeDtypeStruct((2, n_pad, h), jnp.float32),
        mesh=mesh,
        compiler_params=pltpu.CompilerParams(use_tc_tiling_on_sc=False),
        scratch_types=[
            pltpu.VMEM((_K,), jnp.int32),
            pltpu.VMEM((_K,), jnp.int32),
            pltpu.VMEM((_K, h), jnp.float32),
            pltpu.VMEM_SHARED((n_pad, h), jnp.float32),
            pltpu.SemaphoreType.DMA,
        ],
    )
    def agg(g_hbm, src_hbm, dst_hbm, zero_hbm, out_hbm,
            idx_s, idx_d, rows, acc_sh, sem):
        c = lax.axis_index("c")
        s = lax.axis_index("s")
        wid = c * 16 + s
        pltpu.sync_copy(zero_hbm.at[pl.ds(s * rps, rps)],
                        acc_sh.at[pl.ds(s * rps, rps)])
        if not gather:
            pltpu.sync_copy(g_hbm, rows)
        plsc.subcore_barrier()
        base = wid * ept

        def body(i, carry):
            off = base + i * _K
            pltpu.sync_copy(dst_hbm.at[pl.ds(off, _K)], idx_d)
            if gather:
                pltpu.sync_copy(src_hbm.at[pl.ds(off, _K)], idx_s)
                pltpu.async_copy(g_hbm.at[idx_s], rows, sem).wait()
            pltpu.sync_copy(rows, acc_sh.at[idx_d], add=True)
            return carry

        lax.fori_loop(0, nchunks, body, 0)
        plsc.subcore_barrier()
        pltpu.sync_copy(acc_sh.at[pl.ds(s * rps, rps)],
                        out_hbm.at[c, pl.ds(s * rps, rps)])

    return agg


def _dinv(d_ref):
    deg = 1.0 + d_ref[0, :, 0:1] + d_ref[1, :, 0:1]
    return lax.rsqrt(deg)


def _b1_body(x_ref, d_ref, w_ref, o_ref):
    dinv = _dinv(d_ref)
    xw = jnp.dot(x_ref[...], w_ref[...], precision=_HIGH,
                 preferred_element_type=jnp.float32)
    o_ref[...] = xw * dinv


def _mid_body(a_ref, g_ref, d_ref, w_ref, b_ref, o_ref):
    dinv = _dinv(d_ref)
    h = jnp.maximum((a_ref[0] + a_ref[1] + g_ref[...]) * dinv + b_ref[...], 0.0)
    hw = jnp.dot(h, w_ref[...], precision=_HIGH,
                 preferred_element_type=jnp.float32)
    o_ref[...] = hw * dinv


def _head_body(a_ref, g_ref, d_ref, b3_ref, wc1_ref, bc1_ref, wc2_ref, bc2_ref,
               wf1_ref, bf1_ref, wf2_ref, bf2_ref, coords_ref, conf_ref):
    dinv = _dinv(d_ref)
    h3 = jnp.maximum((a_ref[0] + a_ref[1] + g_ref[...]) * dinv + b3_ref[...],
                     0.0)
    c1 = jnp.maximum(jnp.dot(h3, wc1_ref[...], precision=_HIGH,
                             preferred_element_type=jnp.float32) + bc1_ref[...],
                     0.0)
    coords_ref[...] = jnp.dot(c1, wc2_ref[...], precision=_HIGH,
                              preferred_element_type=jnp.float32) + bc2_ref[...]
    f1 = jnp.maximum(jnp.dot(h3, wf1_ref[...], precision=_HIGH,
                             preferred_element_type=jnp.float32) + bf1_ref[...],
                     0.0)
    conf_ref[...] = jax.nn.sigmoid(
        jnp.dot(f1, wf2_ref[...], precision=_HIGH,
                preferred_element_type=jnp.float32) + bf2_ref[...])


def _full(shape):
    return pl.BlockSpec(shape, lambda i: tuple(0 for _ in shape))


def _rows_spec(w):
    return pl.BlockSpec((_R, w), lambda i: (i, 0))


def _acc_spec(w):
    return pl.BlockSpec((2, _R, w), lambda i: (0, i, 0))


def _b1_call(n_pad, d, h):
    return pl.pallas_call(
        _b1_body,
        grid=(n_pad // _R,),
        in_specs=[_rows_spec(d), _acc_spec(_DW), _full((d, h))],
        out_specs=_rows_spec(h),
        out_shape=jax.ShapeDtypeStruct((n_pad, h), jnp.float32),
    )


def _mid_call(n_pad, h_in, h_out):
    return pl.pallas_call(
        _mid_body,
        grid=(n_pad // _R,),
        in_specs=[_acc_spec(h_in), _rows_spec(h_in), _acc_spec(_DW),
                  _full((h_in, h_out)), _full((1, h_in))],
        out_specs=_rows_spec(h_out),
        out_shape=jax.ShapeDtypeStruct((n_pad, h_out), jnp.float32),
    )


def _head_call(n_pad, h3, hm, co, fo):
    return pl.pallas_call(
        _head_body,
        grid=(n_pad // _R,),
        in_specs=[_acc_spec(h3), _rows_spec(h3), _acc_spec(_DW),
                  _full((1, h3)),
                  _full((h3, hm)), _full((1, hm)), _full((hm, co)),
                  _full((1, co)),
                  _full((h3, hm)), _full((1, hm)), _full((hm, fo)),
                  _full((1, fo))],
        out_specs=[_rows_spec(co), _rows_spec(fo)],
        out_shape=[jax.ShapeDtypeStruct((n_pad, co), jnp.float32),
                   jax.ShapeDtypeStruct((n_pad, fo), jnp.float32)],
    )


def kernel(x, edge_index, W1, b1, W2, b2, W3, b3,
           Wc1, bc1, Wc2, bc2, Wf1, bf1, Wf2, bf2):
    n, d = x.shape
    e = edge_index.shape[1]
    h = W1.shape[1]
    h3 = W3.shape[1]
    hm = Wc1.shape[1]
    co = Wc2.shape[1]
    fo = Wf2.shape[1]

    n_pad = -(-n // _R) * _R
    e_pad = -(-e // (_NW * _K)) * (_NW * _K)
    pad_row = n

    ei = edge_index.astype(jnp.int32)
    fill = jnp.full((e_pad - e,), pad_row, jnp.int32)
    src = jnp.concatenate([ei[0], fill])
    dst = jnp.concatenate([ei[1], fill])
    x_pad = jnp.pad(x, ((0, n_pad - n), (0, 0)))

    zeros_h = jnp.zeros((n_pad, h), jnp.float32)
    zeros_h3 = jnp.zeros((n_pad, h3), jnp.float32)
    zeros_dw = jnp.zeros((n_pad, _DW), jnp.float32)
    ones_rows = jnp.ones((_K, _DW), jnp.float32)

    dacc = _sc_agg(n_pad, e_pad, _DW, False)(ones_rows, src, dst, zeros_dw)

    g1 = _b1_call(n_pad, d, h)(x_pad, dacc, W1)
    a1 = _sc_agg(n_pad, e_pad, h, True)(g1, src, dst, zeros_h)
    g2 = _mid_call(n_pad, h, h)(a1, g1, dacc, W2, b1.reshape(1, -1))
    a2 = _sc_agg(n_pad, e_pad, h, True)(g2, src, dst, zeros_h)
    g3 = _mid_call(n_pad, h, h3)(a2, g2, dacc, W3, b2.reshape(1, -1))
    a3 = _sc_agg(n_pad, e_pad, h3, True)(g3, src, dst, zeros_h3)
    coords, conf = _head_call(n_pad, h3, hm, co, fo)(
        a3, g3, dacc, b3.reshape(1, -1),
        Wc1, bc1.reshape(1, -1), Wc2, bc2.reshape(1, -1),
        Wf1, bf1.reshape(1, -1), Wf2, bf2.reshape(1, -1))

    return coords[:n], conf[:n, 0]


# speedup vs baseline: 16.6665x; 1.2454x over previous
import functools

import jax
import jax.numpy as jnp
from jax import lax
from jax.experimental import pallas as pl
from jax.experimental.pallas import tpu as pltpu
from jax.experimental.pallas import tpu_sc as plsc

_K = 128
_NW = 32
_R = 1024
_DW = 16

_HIGH = jax.lax.Precision.HIGHEST


_NB = 4


@functools.lru_cache(maxsize=None)
def _sc_agg(n_pad: int, e_pad: int, h: int, gather: bool):
    ept = e_pad // _NW
    nchunks = ept // _K
    ngroups = nchunks // _NB
    rps = n_pad // 16
    mesh = plsc.VectorSubcoreMesh(core_axis_name="c", subcore_axis_name="s")

    row_scr = [pltpu.VMEM((_K, h), jnp.float32) for _ in range(_NB)]
    sem_scr = [pltpu.SemaphoreType.DMA for _ in range(2 * _NB + 1)]

    @functools.partial(
        pl.kernel,
        out_type=jax.ShapeDtypeStruct((2, n_pad, h), jnp.float32),
        mesh=mesh,
        compiler_params=pltpu.CompilerParams(use_tc_tiling_on_sc=False),
        scratch_types=[
            pltpu.VMEM((nchunks, _K), jnp.int32),
            pltpu.VMEM((nchunks, _K), jnp.int32),
            pltpu.VMEM_SHARED((n_pad, h), jnp.float32),
        ] + row_scr + sem_scr,
    )
    def agg(g_hbm, src_hbm, dst_hbm, zero_hbm, out_hbm,
            idx_s, idx_d, acc_sh, *rest):
        rows = rest[:_NB]
        gsem = rest[_NB:2 * _NB]
        ssem = rest[2 * _NB:3 * _NB]
        isem = rest[3 * _NB]
        c = lax.axis_index("c")
        s = lax.axis_index("s")
        wid = c * 16 + s
        cbase = wid * nchunks
        cp_d = pltpu.async_copy(dst_hbm.at[pl.ds(cbase, nchunks)], idx_d, isem)
        if gather:
            cp_s = pltpu.async_copy(src_hbm.at[pl.ds(cbase, nchunks)],
                                    idx_s, isem)
        pltpu.sync_copy(zero_hbm.at[pl.ds(s * rps, rps)],
                        acc_sh.at[pl.ds(s * rps, rps)])
        cp_d.wait()
        if gather:
            cp_s.wait()
        if not gather:
            pltpu.sync_copy(g_hbm, rows[0])
        plsc.subcore_barrier()

        def start_g(i, b):
            pltpu.async_copy(g_hbm.at[idx_s.at[i]], rows[b], gsem[b])

        def wait_g(b):
            pltpu.make_async_copy(g_hbm.at[idx_s.at[0]], rows[b],
                                  gsem[b]).wait()

        def start_s(i, b):
            pltpu.async_copy(rows[b], acc_sh.at[idx_d.at[i]], ssem[b],
                             add=True)

        def wait_s(b):
            pltpu.make_async_copy(rows[b], acc_sh.at[idx_d.at[0]],
                                  ssem[b]).wait()

        if gather:
            for b in range(_NB):
                start_g(b, b)

            def body(j, carry):
                for b in range(_NB):
                    i = j * _NB + b
                    wait_g(b)
                    start_s(i, b)
                    wait_s(b)
                    start_g(i + _NB, b)
                return carry

            lax.fori_loop(0, ngroups - 1, body, 0)
            tail = (ngroups - 1) * _NB
            for b in range(_NB):
                wait_g(b)
                start_s(tail + b, b)
            for b in range(_NB):
                wait_s(b)
        else:
            def body(j, carry):
                for b in range(_NB):
                    start_s(j * _NB + b, 0)
                for b in range(_NB):
                    wait_s(0)
                return carry

            lax.fori_loop(0, ngroups, body, 0)

        plsc.subcore_barrier()
        pltpu.sync_copy(acc_sh.at[pl.ds(s * rps, rps)],
                        out_hbm.at[c, pl.ds(s * rps, rps)])

    return agg


def _dinv(d_ref):
    deg = 1.0 + d_ref[0, :, 0:1] + d_ref[1, :, 0:1]
    return lax.rsqrt(deg)


def _b1_body(x_ref, d_ref, w_ref, o_ref):
    dinv = _dinv(d_ref)
    xw = jnp.dot(x_ref[...], w_ref[...], precision=_HIGH,
                 preferred_element_type=jnp.float32)
    o_ref[...] = xw * dinv


def _mid_body(a_ref, g_ref, d_ref, w_ref, b_ref, o_ref):
    dinv = _dinv(d_ref)
    h = jnp.maximum((a_ref[0] + a_ref[1] + g_ref[...]) * dinv + b_ref[...], 0.0)
    hw = jnp.dot(h, w_ref[...], precision=_HIGH,
                 preferred_element_type=jnp.float32)
    o_ref[...] = hw * dinv


def _head_body(a_ref, g_ref, d_ref, b3_ref, wc1_ref, bc1_ref, wc2_ref, bc2_ref,
               wf1_ref, bf1_ref, wf2_ref, bf2_ref, coords_ref, conf_ref):
    dinv = _dinv(d_ref)
    h3 = jnp.maximum((a_ref[0] + a_ref[1] + g_ref[...]) * dinv + b3_ref[...],
                     0.0)
    c1 = jnp.maximum(jnp.dot(h3, wc1_ref[...], precision=_HIGH,
                             preferred_element_type=jnp.float32) + bc1_ref[...],
                     0.0)
    coords_ref[...] = jnp.dot(c1, wc2_ref[...], precision=_HIGH,
                              preferred_element_type=jnp.float32) + bc2_ref[...]
    f1 = jnp.maximum(jnp.dot(h3, wf1_ref[...], precision=_HIGH,
                             preferred_element_type=jnp.float32) + bf1_ref[...],
                     0.0)
    conf_ref[...] = jax.nn.sigmoid(
        jnp.dot(f1, wf2_ref[...], precision=_HIGH,
                preferred_element_type=jnp.float32) + bf2_ref[...])


def _full(shape):
    return pl.BlockSpec(shape, lambda i: tuple(0 for _ in shape))


def _rows_spec(w):
    return pl.BlockSpec((_R, w), lambda i: (i, 0))


def _acc_spec(w):
    return pl.BlockSpec((2, _R, w), lambda i: (0, i, 0))


def _b1_call(n_pad, d, h):
    return pl.pallas_call(
        _b1_body,
        grid=(n_pad // _R,),
        in_specs=[_rows_spec(d), _acc_spec(_DW), _full((d, h))],
        out_specs=_rows_spec(h),
        out_shape=jax.ShapeDtypeStruct((n_pad, h), jnp.float32),
    )


def _mid_call(n_pad, h_in, h_out):
    return pl.pallas_call(
        _mid_body,
        grid=(n_pad // _R,),
        in_specs=[_acc_spec(h_in), _rows_spec(h_in), _acc_spec(_DW),
                  _full((h_in, h_out)), _full((1, h_in))],
        out_specs=_rows_spec(h_out),
        out_shape=jax.ShapeDtypeStruct((n_pad, h_out), jnp.float32),
    )


def _head_call(n_pad, h3, hm, co, fo):
    return pl.pallas_call(
        _head_body,
        grid=(n_pad // _R,),
        in_specs=[_acc_spec(h3), _rows_spec(h3), _acc_spec(_DW),
                  _full((1, h3)),
                  _full((h3, hm)), _full((1, hm)), _full((hm, co)),
                  _full((1, co)),
                  _full((h3, hm)), _full((1, hm)), _full((hm, fo)),
                  _full((1, fo))],
        out_specs=[_rows_spec(co), _rows_spec(fo)],
        out_shape=[jax.ShapeDtypeStruct((n_pad, co), jnp.float32),
                   jax.ShapeDtypeStruct((n_pad, fo), jnp.float32)],
    )


def kernel(x, edge_index, W1, b1, W2, b2, W3, b3,
           Wc1, bc1, Wc2, bc2, Wf1, bf1, Wf2, bf2):
    n, d = x.shape
    e = edge_index.shape[1]
    h = W1.shape[1]
    h3 = W3.shape[1]
    hm = Wc1.shape[1]
    co = Wc2.shape[1]
    fo = Wf2.shape[1]

    n_pad = -(-n // _R) * _R
    e_grain = _NW * _K * _NB
    e_pad = -(-e // e_grain) * e_grain
    pad_row = n

    ei = edge_index.astype(jnp.int32)
    fill = jnp.full((e_pad - e,), pad_row, jnp.int32)
    src = jnp.concatenate([ei[0], fill]).reshape(-1, _K)
    dst = jnp.concatenate([ei[1], fill]).reshape(-1, _K)
    x_pad = jnp.pad(x, ((0, n_pad - n), (0, 0)))

    zeros_h = jnp.zeros((n_pad, h), jnp.float32)
    zeros_h3 = jnp.zeros((n_pad, h3), jnp.float32)
    zeros_dw = jnp.zeros((n_pad, _DW), jnp.float32)
    ones_rows = jnp.ones((_K, _DW), jnp.float32)

    dacc = _sc_agg(n_pad, e_pad, _DW, False)(ones_rows, src, dst, zeros_dw)

    g1 = _b1_call(n_pad, d, h)(x_pad, dacc, W1)
    a1 = _sc_agg(n_pad, e_pad, h, True)(g1, src, dst, zeros_h)
    g2 = _mid_call(n_pad, h, h)(a1, g1, dacc, W2, b1.reshape(1, -1))
    a2 = _sc_agg(n_pad, e_pad, h, True)(g2, src, dst, zeros_h)
    g3 = _mid_call(n_pad, h, h3)(a2, g2, dacc, W3, b2.reshape(1, -1))
    a3 = _sc_agg(n_pad, e_pad, h3, True)(g3, src, dst, zeros_h3)
    coords, conf = _head_call(n_pad, h3, hm, co, fo)(
        a3, g3, dacc, b3.reshape(1, -1),
        Wc1, bc1.reshape(1, -1), Wc2, bc2.reshape(1, -1),
        Wf1, bf1.reshape(1, -1), Wf2, bf2.reshape(1, -1))

    return coords[:n], conf[:n, 0]


# speedup vs baseline: 37.1707x; 2.2303x over previous
import functools

import jax
import jax.numpy as jnp
from jax import lax
from jax.experimental import pallas as pl
from jax.experimental.pallas import tpu as pltpu
from jax.experimental.pallas import tpu_sc as plsc

_K = 128
_NW = 32
_R = 1024
_DW = 16

_HIGH = jax.lax.Precision.HIGHEST


_NB = 4


@functools.lru_cache(maxsize=None)
def _sc_agg(n_pad: int, e_pad: int, h: int, gather: bool):
    ept = e_pad // _NW
    nchunks = ept // _K
    ngroups = nchunks // _NB
    rps = n_pad // 16
    mesh = plsc.VectorSubcoreMesh(core_axis_name="c", subcore_axis_name="s")

    row_scr = [pltpu.VMEM((_K, h), jnp.float32) for _ in range(_NB)]
    sem_scr = [pltpu.SemaphoreType.DMA for _ in range(2 * _NB + 1)]

    @functools.partial(
        pl.kernel,
        out_type=jax.ShapeDtypeStruct((2, n_pad, h), jnp.float32),
        mesh=mesh,
        compiler_params=pltpu.CompilerParams(use_tc_tiling_on_sc=False),
        scratch_types=[
            pltpu.VMEM((nchunks, _K), jnp.int32),
            pltpu.VMEM((nchunks, _K), jnp.int32),
            pltpu.VMEM_SHARED((n_pad, h), jnp.float32),
        ] + row_scr + sem_scr,
    )
    def agg(g_hbm, src_hbm, dst_hbm, zero_hbm, out_hbm,
            idx_s, idx_d, acc_sh, *rest):
        rows = rest[:_NB]
        gsem = rest[_NB:2 * _NB]
        ssem = rest[2 * _NB:3 * _NB]
        isem = rest[3 * _NB]
        c = lax.axis_index("c")
        s = lax.axis_index("s")
        wid = c * 16 + s
        cbase = wid * nchunks
        cp_d = pltpu.async_copy(dst_hbm.at[pl.ds(cbase, nchunks)], idx_d, isem)
        if gather:
            cp_s = pltpu.async_copy(src_hbm.at[pl.ds(cbase, nchunks)],
                                    idx_s, isem)
        pltpu.sync_copy(zero_hbm.at[pl.ds(s * rps, rps)],
                        acc_sh.at[pl.ds(s * rps, rps)])
        cp_d.wait()
        if gather:
            cp_s.wait()
        if not gather:
            pltpu.sync_copy(g_hbm, rows[0])
        plsc.subcore_barrier()

        def start_g(i, b):
            pltpu.async_copy(g_hbm.at[idx_s.at[i]], rows[b], gsem[b])

        def wait_g(b):
            pltpu.make_async_copy(g_hbm.at[idx_s.at[0]], rows[b],
                                  gsem[b]).wait()

        def start_s(i, b):
            pltpu.async_copy(rows[b], acc_sh.at[idx_d.at[i]], ssem[b],
                             add=True)

        def wait_s(b):
            pltpu.make_async_copy(rows[b], acc_sh.at[idx_d.at[0]],
                                  ssem[b]).wait()

        if gather:
            for b in range(_NB):
                start_g(b, b)

            def body(j, carry):
                for b in range(_NB):
                    i = j * _NB + b
                    wait_g(b)
                    start_s(i, b)
                    wait_s(b)
                    start_g(i + _NB, b)
                return carry

            lax.fori_loop(0, ngroups - 1, body, 0)
            tail = (ngroups - 1) * _NB
            for b in range(_NB):
                wait_g(b)
                start_s(tail + b, b)
            for b in range(_NB):
                wait_s(b)
        else:
            def body(j, carry):
                for b in range(_NB):
                    start_s(j * _NB + b, 0)
                for b in range(_NB):
                    wait_s(0)
                return carry

            lax.fori_loop(0, ngroups, body, 0)

        plsc.subcore_barrier()
        pltpu.sync_copy(acc_sh.at[pl.ds(s * rps, rps)],
                        out_hbm.at[c, pl.ds(s * rps, rps)])

    return agg


def _dinv(d_ref):
    deg = 1.0 + d_ref[0, :, 0:1] + d_ref[1, :, 0:1]
    return lax.rsqrt(deg)


def _b1_body(x_ref, d_ref, w_ref, o_ref):
    dinv = _dinv(d_ref)
    xw = jnp.dot(x_ref[...], w_ref[...], precision=_HIGH,
                 preferred_element_type=jnp.float32)
    o_ref[...] = xw * dinv


def _mid_body(a_ref, g_ref, d_ref, w_ref, b_ref, o_ref):
    dinv = _dinv(d_ref)
    h = jnp.maximum((a_ref[0] + a_ref[1] + g_ref[...]) * dinv + b_ref[...], 0.0)
    hw = jnp.dot(h, w_ref[...], precision=_HIGH,
                 preferred_element_type=jnp.float32)
    o_ref[...] = hw * dinv


def _head_body(a_ref, g_ref, d_ref, b3_ref, wc1_ref, bc1_ref, wc2_ref, bc2_ref,
               wf1_ref, bf1_ref, wf2_ref, bf2_ref, coords_ref, conf_ref):
    dinv = _dinv(d_ref)
    h3 = jnp.maximum((a_ref[0] + a_ref[1] + g_ref[...]) * dinv + b3_ref[...],
                     0.0)
    c1 = jnp.maximum(jnp.dot(h3, wc1_ref[...], precision=_HIGH,
                             preferred_element_type=jnp.float32) + bc1_ref[...],
                     0.0)
    coords_ref[...] = jnp.dot(c1, wc2_ref[...], precision=_HIGH,
                              preferred_element_type=jnp.float32) + bc2_ref[...]
    f1 = jnp.maximum(jnp.dot(h3, wf1_ref[...], precision=_HIGH,
                             preferred_element_type=jnp.float32) + bf1_ref[...],
                     0.0)
    conf_ref[...] = jax.nn.sigmoid(
        jnp.dot(f1, wf2_ref[...], precision=_HIGH,
                preferred_element_type=jnp.float32) + bf2_ref[...])


def _full(shape):
    return pl.BlockSpec(shape, lambda i: tuple(0 for _ in shape))


def _rows_spec(w):
    return pl.BlockSpec((_R, w), lambda i: (i, 0))


def _acc_spec(w):
    return pl.BlockSpec((2, _R, w), lambda i: (0, i, 0))


def _b1_call(n_pad, d, h):
    return pl.pallas_call(
        _b1_body,
        grid=(n_pad // _R,),
        in_specs=[_rows_spec(d), _acc_spec(_DW), _full((d, h))],
        out_specs=_rows_spec(h),
        out_shape=jax.ShapeDtypeStruct((n_pad, h), jnp.float32),
    )


def _mid_call(n_pad, h_in, h_out):
    return pl.pallas_call(
        _mid_body,
        grid=(n_pad // _R,),
        in_specs=[_acc_spec(h_in), _rows_spec(h_in), _acc_spec(_DW),
                  _full((h_in, h_out)), _full((1, h_in))],
        out_specs=_rows_spec(h_out),
        out_shape=jax.ShapeDtypeStruct((n_pad, h_out), jnp.float32),
    )


def _head_call(n_pad, h3, hm, co, fo):
    return pl.pallas_call(
        _head_body,
        grid=(n_pad // _R,),
        in_specs=[_acc_spec(h3), _rows_spec(h3), _acc_spec(_DW),
                  _full((1, h3)),
                  _full((h3, hm)), _full((1, hm)), _full((hm, co)),
                  _full((1, co)),
                  _full((h3, hm)), _full((1, hm)), _full((hm, fo)),
                  _full((1, fo))],
        out_specs=[_rows_spec(co), _rows_spec(fo)],
        out_shape=[jax.ShapeDtypeStruct((n_pad, co), jnp.float32),
                   jax.ShapeDtypeStruct((n_pad, fo), jnp.float32)],
    )


def kernel(x, edge_index, W1, b1, W2, b2, W3, b3,
           Wc1, bc1, Wc2, bc2, Wf1, bf1, Wf2, bf2):
    n, d = x.shape
    e = edge_index.shape[1]
    h = W1.shape[1]
    h3 = W3.shape[1]
    hm = Wc1.shape[1]
    co = Wc2.shape[1]
    fo = Wf2.shape[1]

    n_pad = -(-(n + 1) // _R) * _R
    e_grain = _NW * _K * _NB
    e_pad = -(-e // e_grain) * e_grain

    ei = edge_index.astype(jnp.int32)
    fill = n + jnp.arange(e_pad - e, dtype=jnp.int32) % (n_pad - n)
    src = jnp.concatenate([ei[0], fill]).reshape(-1, _K)
    dst = jnp.concatenate([ei[1], fill]).reshape(-1, _K)
    x_pad = jnp.pad(x, ((0, n_pad - n), (0, 0)))

    zeros_h = jnp.zeros((n_pad, h), jnp.float32)
    zeros_h3 = jnp.zeros((n_pad, h3), jnp.float32)
    zeros_dw = jnp.zeros((n_pad, _DW), jnp.float32)
    ones_rows = jnp.ones((_K, _DW), jnp.float32)

    dacc = _sc_agg(n_pad, e_pad, _DW, False)(ones_rows, src, dst, zeros_dw)

    g1 = _b1_call(n_pad, d, h)(x_pad, dacc, W1)
    a1 = _sc_agg(n_pad, e_pad, h, True)(g1, src, dst, zeros_h)
    g2 = _mid_call(n_pad, h, h)(a1, g1, dacc, W2, b1.reshape(1, -1))
    a2 = _sc_agg(n_pad, e_pad, h, True)(g2, src, dst, zeros_h)
    g3 = _mid_call(n_pad, h, h3)(a2, g2, dacc, W3, b2.reshape(1, -1))
    a3 = _sc_agg(n_pad, e_pad, h3, True)(g3, src, dst, zeros_h3)
    coords, conf = _head_call(n_pad, h3, hm, co, fo)(
        a3, g3, dacc, b3.reshape(1, -1),
        Wc1, bc1.reshape(1, -1), Wc2, bc2.reshape(1, -1),
        Wf1, bf1.reshape(1, -1), Wf2, bf2.reshape(1, -1))

    return coords[:n], conf[:n, 0]


# speedup vs baseline: 40.6028x; 1.0923x over previous
import functools

import jax
import jax.numpy as jnp
from jax import lax
from jax.experimental import pallas as pl
from jax.experimental.pallas import tpu as pltpu
from jax.experimental.pallas import tpu_sc as plsc

_K = 128
_NW = 32
_R = 1024
_DW = 16

_PREC = jax.lax.Precision.DEFAULT


_NB = 4


@functools.lru_cache(maxsize=None)
def _sc_agg(n_pad: int, e_pad: int, h: int, gather: bool):
    ept = e_pad // _NW
    nchunks = ept // _K
    ngroups = nchunks // _NB
    rps = n_pad // 16
    mesh = plsc.VectorSubcoreMesh(core_axis_name="c", subcore_axis_name="s")

    row_scr = [pltpu.VMEM((_K, h), jnp.float32) for _ in range(_NB)]
    sem_scr = [pltpu.SemaphoreType.DMA for _ in range(2 * _NB + 1)]

    @functools.partial(
        pl.kernel,
        out_type=jax.ShapeDtypeStruct((2, n_pad, h), jnp.float32),
        mesh=mesh,
        compiler_params=pltpu.CompilerParams(use_tc_tiling_on_sc=False),
        scratch_types=[
            pltpu.VMEM((nchunks, _K), jnp.int32),
            pltpu.VMEM((nchunks, _K), jnp.int32),
            pltpu.VMEM_SHARED((n_pad, h), jnp.float32),
        ] + row_scr + sem_scr,
    )
    def agg(g_hbm, src_hbm, dst_hbm, zero_hbm, out_hbm,
            idx_s, idx_d, acc_sh, *rest):
        rows = rest[:_NB]
        gsem = rest[_NB:2 * _NB]
        ssem = rest[2 * _NB:3 * _NB]
        isem = rest[3 * _NB]
        c = lax.axis_index("c")
        s = lax.axis_index("s")
        wid = c * 16 + s
        cbase = wid * nchunks
        cp_d = pltpu.async_copy(dst_hbm.at[pl.ds(cbase, nchunks)], idx_d, isem)
        if gather:
            cp_s = pltpu.async_copy(src_hbm.at[pl.ds(cbase, nchunks)],
                                    idx_s, isem)
        pltpu.sync_copy(zero_hbm.at[pl.ds(s * rps, rps)],
                        acc_sh.at[pl.ds(s * rps, rps)])
        cp_d.wait()
        if gather:
            cp_s.wait()
        if not gather:
            pltpu.sync_copy(g_hbm, rows[0])
        plsc.subcore_barrier()

        def start_g(i, b):
            pltpu.async_copy(g_hbm.at[idx_s.at[i]], rows[b], gsem[b])

        def wait_g(b):
            pltpu.make_async_copy(g_hbm.at[idx_s.at[0]], rows[b],
                                  gsem[b]).wait()

        def start_s(i, b):
            pltpu.async_copy(rows[b], acc_sh.at[idx_d.at[i]], ssem[b],
                             add=True)

        def wait_s(b):
            pltpu.make_async_copy(rows[b], acc_sh.at[idx_d.at[0]],
                                  ssem[b]).wait()

        if gather:
            for b in range(_NB):
                start_g(b, b)

            def body(j, carry):
                for b in range(_NB):
                    i = j * _NB + b
                    wait_g(b)
                    start_s(i, b)
                    wait_s(b)
                    start_g(i + _NB, b)
                return carry

            lax.fori_loop(0, ngroups - 1, body, 0)
            tail = (ngroups - 1) * _NB
            for b in range(_NB):
                wait_g(b)
                start_s(tail + b, b)
            for b in range(_NB):
                wait_s(b)
        else:
            def body(j, carry):
                for b in range(_NB):
                    start_s(j * _NB + b, 0)
                for b in range(_NB):
                    wait_s(0)
                return carry

            lax.fori_loop(0, ngroups, body, 0)

        plsc.subcore_barrier()
        pltpu.sync_copy(acc_sh.at[pl.ds(s * rps, rps)],
                        out_hbm.at[c, pl.ds(s * rps, rps)])

    return agg


def _dinv(d_ref):
    deg = 1.0 + d_ref[0, :, 0:1] + d_ref[1, :, 0:1]
    return lax.rsqrt(deg)


def _b1_body(x_ref, d_ref, w_ref, o_ref):
    dinv = _dinv(d_ref)
    xw = jnp.dot(x_ref[...], w_ref[...], precision=_PREC,
                 preferred_element_type=jnp.float32)
    o_ref[...] = xw * dinv


def _mid_body(a_ref, g_ref, d_ref, w_ref, b_ref, o_ref):
    dinv = _dinv(d_ref)
    h = jnp.maximum((a_ref[0] + a_ref[1] + g_ref[...]) * dinv + b_ref[...], 0.0)
    hw = jnp.dot(h, w_ref[...], precision=_PREC,
                 preferred_element_type=jnp.float32)
    o_ref[...] = hw * dinv


def _head_body(a_ref, g_ref, d_ref, b3_ref, wc1_ref, bc1_ref, wc2_ref, bc2_ref,
               wf1_ref, bf1_ref, wf2_ref, bf2_ref, coords_ref, conf_ref):
    dinv = _dinv(d_ref)
    h3 = jnp.maximum((a_ref[0] + a_ref[1] + g_ref[...]) * dinv + b3_ref[...],
                     0.0)
    c1 = jnp.maximum(jnp.dot(h3, wc1_ref[...], precision=_PREC,
                             preferred_element_type=jnp.float32) + bc1_ref[...],
                     0.0)
    coords_ref[...] = jnp.dot(c1, wc2_ref[...], precision=_PREC,
                              preferred_element_type=jnp.float32) + bc2_ref[...]
    f1 = jnp.maximum(jnp.dot(h3, wf1_ref[...], precision=_PREC,
                             preferred_element_type=jnp.float32) + bf1_ref[...],
                     0.0)
    conf_ref[...] = jax.nn.sigmoid(
        jnp.dot(f1, wf2_ref[...], precision=_PREC,
                preferred_element_type=jnp.float32) + bf2_ref[...])


def _full(shape):
    return pl.BlockSpec(shape, lambda i: tuple(0 for _ in shape))


def _rows_spec(w):
    return pl.BlockSpec((_R, w), lambda i: (i, 0))


def _acc_spec(w):
    return pl.BlockSpec((2, _R, w), lambda i: (0, i, 0))


def _b1_call(n_pad, d, h):
    return pl.pallas_call(
        _b1_body,
        grid=(n_pad // _R,),
        in_specs=[_rows_spec(d), _acc_spec(_DW), _full((d, h))],
        out_specs=_rows_spec(h),
        out_shape=jax.ShapeDtypeStruct((n_pad, h), jnp.float32),
    )


def _mid_call(n_pad, h_in, h_out):
    return pl.pallas_call(
        _mid_body,
        grid=(n_pad // _R,),
        in_specs=[_acc_spec(h_in), _rows_spec(h_in), _acc_spec(_DW),
                  _full((h_in, h_out)), _full((1, h_in))],
        out_specs=_rows_spec(h_out),
        out_shape=jax.ShapeDtypeStruct((n_pad, h_out), jnp.float32),
    )


def _head_call(n_pad, h3, hm, co, fo):
    return pl.pallas_call(
        _head_body,
        grid=(n_pad // _R,),
        in_specs=[_acc_spec(h3), _rows_spec(h3), _acc_spec(_DW),
                  _full((1, h3)),
                  _full((h3, hm)), _full((1, hm)), _full((hm, co)),
                  _full((1, co)),
                  _full((h3, hm)), _full((1, hm)), _full((hm, fo)),
                  _full((1, fo))],
        out_specs=[_rows_spec(co), _rows_spec(fo)],
        out_shape=[jax.ShapeDtypeStruct((n_pad, co), jnp.float32),
                   jax.ShapeDtypeStruct((n_pad, fo), jnp.float32)],
    )


def kernel(x, edge_index, W1, b1, W2, b2, W3, b3,
           Wc1, bc1, Wc2, bc2, Wf1, bf1, Wf2, bf2):
    n, d = x.shape
    e = edge_index.shape[1]
    h = W1.shape[1]
    h3 = W3.shape[1]
    hm = Wc1.shape[1]
    co = Wc2.shape[1]
    fo = Wf2.shape[1]

    n_pad = -(-(n + 1) // _R) * _R
    e_grain = _NW * _K * _NB
    e_pad = -(-e // e_grain) * e_grain

    ei = edge_index.astype(jnp.int32)
    fill = n + jnp.arange(e_pad - e, dtype=jnp.int32) % (n_pad - n)
    src = jnp.concatenate([ei[0], fill]).reshape(-1, _K)
    dst = jnp.concatenate([ei[1], fill]).reshape(-1, _K)
    x_pad = jnp.pad(x, ((0, n_pad - n), (0, 0)))

    zeros_h = jnp.zeros((n_pad, h), jnp.float32)
    zeros_h3 = jnp.zeros((n_pad, h3), jnp.float32)
    zeros_dw = jnp.zeros((n_pad, _DW), jnp.float32)
    ones_rows = jnp.ones((_K, _DW), jnp.float32)

    dacc = _sc_agg(n_pad, e_pad, _DW, False)(ones_rows, src, dst, zeros_dw)

    g1 = _b1_call(n_pad, d, h)(x_pad, dacc, W1)
    a1 = _sc_agg(n_pad, e_pad, h, True)(g1, src, dst, zeros_h)
    g2 = _mid_call(n_pad, h, h)(a1, g1, dacc, W2, b1.reshape(1, -1))
    a2 = _sc_agg(n_pad, e_pad, h, True)(g2, src, dst, zeros_h)
    g3 = _mid_call(n_pad, h, h3)(a2, g2, dacc, W3, b2.reshape(1, -1))
    a3 = _sc_agg(n_pad, e_pad, h3, True)(g3, src, dst, zeros_h3)
    coords, conf = _head_call(n_pad, h3, hm, co, fo)(
        a3, g3, dacc, b3.reshape(1, -1),
        Wc1, bc1.reshape(1, -1), Wc2, bc2.reshape(1, -1),
        Wf1, bf1.reshape(1, -1), Wf2, bf2.reshape(1, -1))

    return coords[:n], conf[:n, 0]


# speedup vs baseline: 42.1243x; 1.0375x over previous
import functools

import jax
import jax.numpy as jnp
from jax import lax
from jax.experimental import pallas as pl
from jax.experimental.pallas import tpu as pltpu
from jax.experimental.pallas import tpu_sc as plsc

_K = 128
_NW = 32
_R = 2048
_DW = 16

_PREC = jax.lax.Precision.DEFAULT


_NB = 8


@functools.lru_cache(maxsize=None)
def _sc_agg(n_pad: int, e_pad: int, h: int, gather: bool):
    ept = e_pad // _NW
    nchunks = ept // _K
    ngroups = nchunks // _NB
    rps = n_pad // 16
    mesh = plsc.VectorSubcoreMesh(core_axis_name="c", subcore_axis_name="s")

    row_scr = [pltpu.VMEM((_K, h), jnp.float32) for _ in range(_NB)]
    sem_scr = [pltpu.SemaphoreType.DMA for _ in range(2 * _NB + 1)]

    @functools.partial(
        pl.kernel,
        out_type=jax.ShapeDtypeStruct((2, n_pad, h), jnp.float32),
        mesh=mesh,
        compiler_params=pltpu.CompilerParams(use_tc_tiling_on_sc=False),
        scratch_types=[
            pltpu.VMEM((nchunks, _K), jnp.int32),
            pltpu.VMEM((nchunks, _K), jnp.int32),
            pltpu.VMEM_SHARED((n_pad, h), jnp.float32),
        ] + row_scr + sem_scr,
    )
    def agg(g_hbm, src_hbm, dst_hbm, zero_hbm, out_hbm,
            idx_s, idx_d, acc_sh, *rest):
        rows = rest[:_NB]
        gsem = rest[_NB:2 * _NB]
        ssem = rest[2 * _NB:3 * _NB]
        isem = rest[3 * _NB]
        c = lax.axis_index("c")
        s = lax.axis_index("s")
        wid = c * 16 + s
        cbase = wid * nchunks
        cp_d = pltpu.async_copy(dst_hbm.at[pl.ds(cbase, nchunks)], idx_d, isem)
        if gather:
            cp_s = pltpu.async_copy(src_hbm.at[pl.ds(cbase, nchunks)],
                                    idx_s, isem)
        pltpu.sync_copy(zero_hbm.at[pl.ds(s * rps, rps)],
                        acc_sh.at[pl.ds(s * rps, rps)])
        cp_d.wait()
        if gather:
            cp_s.wait()
        if not gather:
            pltpu.sync_copy(g_hbm, rows[0])
        plsc.subcore_barrier()

        def start_g(i, b):
            pltpu.async_copy(g_hbm.at[idx_s.at[i]], rows[b], gsem[b])

        def wait_g(b):
            pltpu.make_async_copy(g_hbm.at[idx_s.at[0]], rows[b],
                                  gsem[b]).wait()

        def start_s(i, b):
            pltpu.async_copy(rows[b], acc_sh.at[idx_d.at[i]], ssem[b],
                             add=True)

        def wait_s(b):
            pltpu.make_async_copy(rows[b], acc_sh.at[idx_d.at[0]],
                                  ssem[b]).wait()

        if gather:
            for b in range(_NB):
                start_g(b, b)

            def body(j, carry):
                for b in range(_NB):
                    i = j * _NB + b
                    wait_g(b)
                    start_s(i, b)
                    wait_s(b)
                    start_g(i + _NB, b)
                return carry

            lax.fori_loop(0, ngroups - 1, body, 0)
            tail = (ngroups - 1) * _NB
            for b in range(_NB):
                wait_g(b)
                start_s(tail + b, b)
            for b in range(_NB):
                wait_s(b)
        else:
            def body(j, carry):
                for b in range(_NB):
                    start_s(j * _NB + b, 0)
                for b in range(_NB):
                    wait_s(0)
                return carry

            lax.fori_loop(0, ngroups, body, 0)

        plsc.subcore_barrier()
        pltpu.sync_copy(acc_sh.at[pl.ds(s * rps, rps)],
                        out_hbm.at[c, pl.ds(s * rps, rps)])

    return agg


def _dinv(d_ref):
    deg = 1.0 + d_ref[0, :, 0:1] + d_ref[1, :, 0:1]
    return lax.rsqrt(deg)


def _b1_body(x_ref, d_ref, w_ref, o_ref):
    dinv = _dinv(d_ref)
    xw = jnp.dot(x_ref[...], w_ref[...], precision=_PREC,
                 preferred_element_type=jnp.float32)
    o_ref[...] = xw * dinv


def _mid_body(a_ref, g_ref, d_ref, w_ref, b_ref, o_ref):
    dinv = _dinv(d_ref)
    h = jnp.maximum((a_ref[0] + a_ref[1] + g_ref[...]) * dinv + b_ref[...], 0.0)
    hw = jnp.dot(h, w_ref[...], precision=_PREC,
                 preferred_element_type=jnp.float32)
    o_ref[...] = hw * dinv


def _head_body(a_ref, g_ref, d_ref, b3_ref, wc1_ref, bc1_ref, wc2_ref, bc2_ref,
               wf1_ref, bf1_ref, wf2_ref, bf2_ref, coords_ref, conf_ref):
    dinv = _dinv(d_ref)
    h3 = jnp.maximum((a_ref[0] + a_ref[1] + g_ref[...]) * dinv + b3_ref[...],
                     0.0)
    c1 = jnp.maximum(jnp.dot(h3, wc1_ref[...], precision=_PREC,
                             preferred_element_type=jnp.float32) + bc1_ref[...],
                     0.0)
    coords_ref[...] = jnp.dot(c1, wc2_ref[...], precision=_PREC,
                              preferred_element_type=jnp.float32) + bc2_ref[...]
    f1 = jnp.maximum(jnp.dot(h3, wf1_ref[...], precision=_PREC,
                             preferred_element_type=jnp.float32) + bf1_ref[...],
                     0.0)
    conf_ref[...] = jax.nn.sigmoid(
        jnp.dot(f1, wf2_ref[...], precision=_PREC,
                preferred_element_type=jnp.float32) + bf2_ref[...])


def _full(shape):
    return pl.BlockSpec(shape, lambda i: tuple(0 for _ in shape))


def _rows_spec(w):
    return pl.BlockSpec((_R, w), lambda i: (i, 0))


def _acc_spec(w):
    return pl.BlockSpec((2, _R, w), lambda i: (0, i, 0))


def _b1_call(n_pad, d, h):
    return pl.pallas_call(
        _b1_body,
        grid=(n_pad // _R,),
        in_specs=[_rows_spec(d), _acc_spec(_DW), _full((d, h))],
        out_specs=_rows_spec(h),
        out_shape=jax.ShapeDtypeStruct((n_pad, h), jnp.float32),
    )


def _mid_call(n_pad, h_in, h_out):
    return pl.pallas_call(
        _mid_body,
        grid=(n_pad // _R,),
        in_specs=[_acc_spec(h_in), _rows_spec(h_in), _acc_spec(_DW),
                  _full((h_in, h_out)), _full((1, h_in))],
        out_specs=_rows_spec(h_out),
        out_shape=jax.ShapeDtypeStruct((n_pad, h_out), jnp.float32),
    )


def _head_call(n_pad, h3, hm, co, fo):
    return pl.pallas_call(
        _head_body,
        grid=(n_pad // _R,),
        in_specs=[_acc_spec(h3), _rows_spec(h3), _acc_spec(_DW),
                  _full((1, h3)),
                  _full((h3, hm)), _full((1, hm)), _full((hm, co)),
                  _full((1, co)),
                  _full((h3, hm)), _full((1, hm)), _full((hm, fo)),
                  _full((1, fo))],
        out_specs=[_rows_spec(co), _rows_spec(fo)],
        out_shape=[jax.ShapeDtypeStruct((n_pad, co), jnp.float32),
                   jax.ShapeDtypeStruct((n_pad, fo), jnp.float32)],
    )


def kernel(x, edge_index, W1, b1, W2, b2, W3, b3,
           Wc1, bc1, Wc2, bc2, Wf1, bf1, Wf2, bf2):
    n, d = x.shape
    e = edge_index.shape[1]
    h = W1.shape[1]
    h3 = W3.shape[1]
    hm = Wc1.shape[1]
    co = Wc2.shape[1]
    fo = Wf2.shape[1]

    n_pad = -(-(n + 1) // _R) * _R
    e_grain = _NW * _K * _NB
    e_pad = -(-e // e_grain) * e_grain

    ei = edge_index.astype(jnp.int32)
    fill = n + jnp.arange(e_pad - e, dtype=jnp.int32) % (n_pad - n)
    src = jnp.concatenate([ei[0], fill]).reshape(-1, _K)
    dst = jnp.concatenate([ei[1], fill]).reshape(-1, _K)
    x_pad = jnp.pad(x, ((0, n_pad - n), (0, 0)))

    zeros_h = jnp.zeros((n_pad, h), jnp.float32)
    zeros_h3 = jnp.zeros((n_pad, h3), jnp.float32)
    zeros_dw = jnp.zeros((n_pad, _DW), jnp.float32)
    ones_rows = jnp.ones((_K, _DW), jnp.float32)

    dacc = _sc_agg(n_pad, e_pad, _DW, False)(ones_rows, src, dst, zeros_dw)

    g1 = _b1_call(n_pad, d, h)(x_pad, dacc, W1)
    a1 = _sc_agg(n_pad, e_pad, h, True)(g1, src, dst, zeros_h)
    g2 = _mid_call(n_pad, h, h)(a1, g1, dacc, W2, b1.reshape(1, -1))
    a2 = _sc_agg(n_pad, e_pad, h, True)(g2, src, dst, zeros_h)
    g3 = _mid_call(n_pad, h, h3)(a2, g2, dacc, W3, b2.reshape(1, -1))
    a3 = _sc_agg(n_pad, e_pad, h3, True)(g3, src, dst, zeros_h3)
    coords, conf = _head_call(n_pad, h3, hm, co, fo)(
        a3, g3, dacc, b3.reshape(1, -1),
        Wc1, bc1.reshape(1, -1), Wc2, bc2.reshape(1, -1),
        Wf1, bf1.reshape(1, -1), Wf2, bf2.reshape(1, -1))

    return coords[:n], conf[:n, 0]


# speedup vs baseline: 42.6986x; 1.0136x over previous
import functools

import jax
import jax.numpy as jnp
from jax import lax
from jax.experimental import pallas as pl
from jax.experimental.pallas import tpu as pltpu
from jax.experimental.pallas import tpu_sc as plsc

_K = 128
_NW = 32
_R = 2048
_DW = 16

_PREC = jax.lax.Precision.DEFAULT


_NB = 8


@functools.lru_cache(maxsize=None)
def _sc_agg(n_pad: int, e_pad: int, h: int, gather: bool):
    ept = e_pad // _NW
    nchunks = ept // _K
    ngroups = nchunks // _NB
    rps = n_pad // 16
    mesh = plsc.VectorSubcoreMesh(core_axis_name="c", subcore_axis_name="s")

    row_scr = [pltpu.VMEM((_K, h), jnp.float32) for _ in range(_NB)]
    sem_scr = [pltpu.SemaphoreType.DMA for _ in range(2 * _NB + 1)]

    @functools.partial(
        pl.kernel,
        out_type=jax.ShapeDtypeStruct((2, n_pad, h), jnp.float32),
        mesh=mesh,
        compiler_params=pltpu.CompilerParams(use_tc_tiling_on_sc=False),
        scratch_types=[
            pltpu.VMEM((nchunks, _K), jnp.int32),
            pltpu.VMEM((nchunks, _K), jnp.int32),
            pltpu.VMEM_SHARED((n_pad, h), jnp.float32),
        ] + row_scr + sem_scr,
    )
    def agg(g_hbm, src_hbm, dst_hbm, zero_hbm, out_hbm,
            idx_s, idx_d, acc_sh, *rest):
        rows = rest[:_NB]
        gsem = rest[_NB:2 * _NB]
        ssem = rest[2 * _NB:3 * _NB]
        isem = rest[3 * _NB]
        c = lax.axis_index("c")
        s = lax.axis_index("s")
        wid = c * 16 + s
        cbase = wid * nchunks
        cp_d = pltpu.async_copy(dst_hbm.at[pl.ds(cbase, nchunks)], idx_d, isem)
        if gather:
            cp_s = pltpu.async_copy(src_hbm.at[pl.ds(cbase, nchunks)],
                                    idx_s, isem)
        pltpu.sync_copy(zero_hbm.at[pl.ds(s * rps, rps)],
                        acc_sh.at[pl.ds(s * rps, rps)])
        cp_d.wait()
        if gather:
            cp_s.wait()
        if not gather:
            pltpu.sync_copy(g_hbm, rows[0])
        plsc.subcore_barrier()

        def start_g(i, b):
            pltpu.async_copy(g_hbm.at[idx_s.at[i]], rows[b], gsem[b])

        def wait_g(b):
            pltpu.make_async_copy(g_hbm.at[idx_s.at[0]], rows[b],
                                  gsem[b]).wait()

        def start_s(i, b):
            pltpu.async_copy(rows[b], acc_sh.at[idx_d.at[i]], ssem[b],
                             add=True)

        def wait_s(b):
            pltpu.make_async_copy(rows[b], acc_sh.at[idx_d.at[0]],
                                  ssem[b]).wait()

        if gather:
            for b in range(_NB):
                start_g(b, b)

            def body(j, carry):
                for b in range(_NB):
                    i = j * _NB + b
                    wait_g(b)
                    start_s(i, b)
                    wait_s(b)
                    start_g(i + _NB, b)
                return carry

            lax.fori_loop(0, ngroups - 1, body, 0)
            tail = (ngroups - 1) * _NB
            for b in range(_NB):
                wait_g(b)
                start_s(tail + b, b)
            for b in range(_NB):
                wait_s(b)
        else:
            def body(j, carry):
                for b in range(_NB):
                    start_s(j * _NB + b, 0)
                for b in range(_NB):
                    wait_s(0)
                return carry

            lax.fori_loop(0, ngroups, body, 0)

        plsc.subcore_barrier()
        pltpu.sync_copy(acc_sh.at[pl.ds(s * rps, rps)],
                        out_hbm.at[c, pl.ds(s * rps, rps)])

    return agg


def _dinv(d_ref):
    deg = 1.0 + d_ref[0, :, 0:1] + d_ref[1, :, 0:1]
    return lax.rsqrt(deg)


def _b1_body(x_ref, d_ref, w_ref, o_ref):
    dinv = _dinv(d_ref)
    xw = jnp.dot(x_ref[...], w_ref[...], precision=_PREC,
                 preferred_element_type=jnp.float32)
    o_ref[...] = xw * dinv


def _mid_body(a_ref, g_ref, d_ref, w_ref, b_ref, o_ref):
    dinv = _dinv(d_ref)
    h = jnp.maximum((a_ref[0] + a_ref[1] + g_ref[...]) * dinv + b_ref[...], 0.0)
    hw = jnp.dot(h, w_ref[...], precision=_PREC,
                 preferred_element_type=jnp.float32)
    o_ref[...] = hw * dinv


def _head_body(a_ref, g_ref, d_ref, b3_ref, wc1_ref, bc1_ref, wc2_ref, bc2_ref,
               wf1_ref, bf1_ref, wf2_ref, bf2_ref, coords_ref, conf_ref):
    dinv = _dinv(d_ref)
    h3 = jnp.maximum((a_ref[0] + a_ref[1] + g_ref[...]) * dinv + b3_ref[...],
                     0.0)
    c1 = jnp.maximum(jnp.dot(h3, wc1_ref[...], precision=_PREC,
                             preferred_element_type=jnp.float32) + bc1_ref[...],
                     0.0)
    coords_ref[...] = jnp.dot(c1, wc2_ref[...], precision=_PREC,
                              preferred_element_type=jnp.float32) + bc2_ref[...]
    f1 = jnp.maximum(jnp.dot(h3, wf1_ref[...], precision=_PREC,
                             preferred_element_type=jnp.float32) + bf1_ref[...],
                     0.0)
    conf_ref[...] = jax.nn.sigmoid(
        jnp.dot(f1, wf2_ref[...], precision=_PREC,
                preferred_element_type=jnp.float32) + bf2_ref[...])


def _full(shape):
    return pl.BlockSpec(shape, lambda i: tuple(0 for _ in shape))


def _rows_spec(w):
    return pl.BlockSpec((_R, w), lambda i: (i, 0))


def _acc_spec(w):
    return pl.BlockSpec((2, _R, w), lambda i: (0, i, 0))


def _b1_call(n_pad, d, h):
    return pl.pallas_call(
        _b1_body,
        grid=(n_pad // _R,),
        in_specs=[_rows_spec(d), _acc_spec(_DW), _full((d, h))],
        out_specs=_rows_spec(h),
        out_shape=jax.ShapeDtypeStruct((n_pad, h), jnp.float32),
    )


def _mid_call(n_pad, h_in, h_out):
    return pl.pallas_call(
        _mid_body,
        grid=(n_pad // _R,),
        in_specs=[_acc_spec(h_in), _rows_spec(h_in), _acc_spec(_DW),
                  _full((h_in, h_out)), _full((1, h_in))],
        out_specs=_rows_spec(h_out),
        out_shape=jax.ShapeDtypeStruct((n_pad, h_out), jnp.float32),
    )


def _head_call(n, n_pad, h3, hm, co, fo):
    return pl.pallas_call(
        _head_body,
        grid=(n_pad // _R,),
        in_specs=[_acc_spec(h3), _rows_spec(h3), _acc_spec(_DW),
                  _full((1, h3)),
                  _full((h3, hm)), _full((1, hm)), _full((hm, co)),
                  _full((1, co)),
                  _full((h3, hm)), _full((1, hm)), _full((hm, fo)),
                  _full((1, fo))],
        out_specs=[_rows_spec(co), _rows_spec(fo)],
        out_shape=[jax.ShapeDtypeStruct((n, co), jnp.float32),
                   jax.ShapeDtypeStruct((n, fo), jnp.float32)],
    )


def kernel(x, edge_index, W1, b1, W2, b2, W3, b3,
           Wc1, bc1, Wc2, bc2, Wf1, bf1, Wf2, bf2):
    n, d = x.shape
    e = edge_index.shape[1]
    h = W1.shape[1]
    h3 = W3.shape[1]
    hm = Wc1.shape[1]
    co = Wc2.shape[1]
    fo = Wf2.shape[1]

    n_pad = -(-(n + 1) // _R) * _R
    e_grain = _NW * _K * _NB
    e_pad = -(-e // e_grain) * e_grain

    ei = edge_index.astype(jnp.int32)
    fill = n + jnp.arange(e_pad - e, dtype=jnp.int32) % (n_pad - n)
    src = jnp.concatenate([ei[0], fill]).reshape(-1, _K)
    dst = jnp.concatenate([ei[1], fill]).reshape(-1, _K)

    zeros_h = jnp.zeros((n_pad, h), jnp.float32)
    zeros_h3 = jnp.zeros((n_pad, h3), jnp.float32)
    zeros_dw = jnp.zeros((n_pad, _DW), jnp.float32)
    ones_rows = jnp.ones((_K, _DW), jnp.float32)

    dacc = _sc_agg(n_pad, e_pad, _DW, False)(ones_rows, src, dst, zeros_dw)

    g1 = _b1_call(n_pad, d, h)(x, dacc, W1)
    a1 = _sc_agg(n_pad, e_pad, h, True)(g1, src, dst, zeros_h)
    g2 = _mid_call(n_pad, h, h)(a1, g1, dacc, W2, b1.reshape(1, -1))
    a2 = _sc_agg(n_pad, e_pad, h, True)(g2, src, dst, zeros_h)
    g3 = _mid_call(n_pad, h, h3)(a2, g2, dacc, W3, b2.reshape(1, -1))
    a3 = _sc_agg(n_pad, e_pad, h3, True)(g3, src, dst, zeros_h3)
    coords, conf = _head_call(n, n_pad, h3, hm, co, fo)(
        a3, g3, dacc, b3.reshape(1, -1),
        Wc1, bc1.reshape(1, -1), Wc2, bc2.reshape(1, -1),
        Wf1, bf1.reshape(1, -1), Wf2, bf2.reshape(1, -1))

    return coords, conf[:, 0]
